# Initial kernel scaffold; baseline (speedup 1.0000x reference)
import jax
import jax.numpy as jnp
from jax.experimental import pallas as pl


def kernel(x, window_idx):
    raise NotImplementedError("write your pallas kernel here")


# speedup vs baseline: 3.4200x; 3.4200x over previous
import jax
import jax.numpy as jnp
from jax.experimental import pallas as pl
from jax.experimental.pallas import tpu as pltpu

_MASK_RATIO = 0.15
_ROWS_PER_BLOCK = 256


def _mask_body(widx_ref, m0_ref, m1_ref, x_ref, o_ref):
    row = jnp.where(widx_ref[0] == 0, m0_ref[...], m1_ref[...])
    o_ref[...] = x_ref[...] * row


def kernel(x, window_idx):
    batch, chans, seq = x.shape
    n_mask = int(seq * _MASK_RATIO)

    perm = jax.random.permutation(jax.random.key(42), seq)
    mask_idx = perm[:n_mask]
    mask0 = jnp.ones((seq,), jnp.float32).at[mask_idx].set(0.0)
    mask1 = jnp.ones((seq,), jnp.float32).at[seq - 1].set(0.0)
    mask0 = mask0.reshape(1, seq)
    mask1 = mask1.reshape(1, seq)

    rows = batch * chans
    x2 = x.reshape(rows, seq)
    widx = jnp.asarray(window_idx, jnp.int32).reshape(1)

    blk = _ROWS_PER_BLOCK
    assert rows % blk == 0

    out = pl.pallas_call(
        _mask_body,
        grid=(rows // blk,),
        in_specs=[
            pl.BlockSpec(memory_space=pltpu.SMEM),
            pl.BlockSpec((1, seq), lambda i: (0, 0)),
            pl.BlockSpec((1, seq), lambda i: (0, 0)),
            pl.BlockSpec((blk, seq), lambda i: (i, 0)),
        ],
        out_specs=pl.BlockSpec((blk, seq), lambda i: (i, 0)),
        out_shape=jax.ShapeDtypeStruct((rows, seq), x.dtype),
        compiler_params=pltpu.CompilerParams(
            dimension_semantics=("arbitrary",),
        ),
    )(widx, mask0, mask1, x2)
    return out.reshape(batch, chans, seq)
